# baseline (speedup 1.0000x reference)
import functools

import jax
import jax.numpy as jnp
from jax import lax
from jax.experimental import pallas as pl
from jax.experimental.pallas import tpu as pltpu
from jax.experimental.pallas import tpu_sc as plsc

N = 4096
C = 1024
H = 2048
E = 8
K = 2
BLK = 256
NPAIR = N * K
NB = NPAIR // BLK + E
PAD_CAP = NB * BLK
BR = 512
NEG = -1e30

NW = 32
GRW = PAD_CAP // NW
GCH = 32
GNB = 3
CW = N // NW
CCH = 16
CNB = 3

def _sc_mesh():
    return plsc.VectorSubcoreMesh(core_axis_name="c", subcore_axis_name="s")


def _router_body(x_ref, wr_ref, idx_ref, w_ref, util_ref, cnt_ref):
    step = pl.program_id(0)
    x = x_ref[...]
    logits = jnp.dot(x, wr_ref[...], preferred_element_type=jnp.float32)
    lane = jax.lax.broadcasted_iota(jnp.int32, (BR, 128), 1)
    valid = lane < E
    lg = jnp.where(valid, logits, NEG)
    m1 = jnp.max(lg, axis=1, keepdims=True)
    a1 = jnp.min(jnp.where(lg == m1, lane, 128), axis=1, keepdims=True)
    lg2 = jnp.where(lane == a1, NEG, lg)
    m2 = jnp.max(lg2, axis=1, keepdims=True)
    a2 = jnp.min(jnp.where(lg2 == m2, lane, 128), axis=1, keepdims=True)
    ex = jnp.where(valid, jnp.exp(lg - m1), 0.0)
    z = jnp.sum(ex, axis=1, keepdims=True)
    p1 = 1.0 / z
    p2 = jnp.exp(m2 - m1) / z
    denom = p1 + p2 + 1e-9
    w1 = p1 / denom
    w2 = p2 / denom
    w_ref[...] = jnp.where(lane == 0, w1, jnp.where(lane == 1, w2, 0.0))

    sel1 = (lane == a1).astype(jnp.int32)
    sel2 = (lane == a2).astype(jnp.int32)
    both = sel1 + sel2
    acc = both
    d = 1
    while d < BR:
        top = jnp.zeros((d, 128), jnp.int32)
        acc = acc + jnp.concatenate([top, acc[:BR - d]], axis=0)
        d *= 2
    cum_ex = acc - both

    @pl.when(step == 0)
    def _():
        util_ref[...] = jnp.zeros_like(util_ref)
        cnt_ref[...] = jnp.zeros_like(cnt_ref)

    carry = cnt_ref[...].astype(jnp.int32)
    rank_base = cum_ex + carry
    rank0 = jnp.sum(jnp.where(lane == a1, rank_base, 0), axis=1, keepdims=True)
    rank1 = jnp.sum(jnp.where(lane == a2, rank_base, 0), axis=1, keepdims=True)
    idx_ref[...] = jnp.where(
        lane == 0, a1,
        jnp.where(lane == 1, a2,
                  jnp.where(lane == 2, rank0,
                            jnp.where(lane == 3, rank1, 0))))

    cnt = jnp.sum(both.astype(jnp.float32), axis=0, keepdims=True)
    util_ref[...] += cnt
    cnt_ref[...] += cnt

    @pl.when(step == (N // BR) - 1)
    def _():
        c = util_ref[...]
        util_ref[...] = c / (jnp.sum(c) + 1e-9)


def _router(xf, wr_pad):
    return pl.pallas_call(
        _router_body,
        grid=(N // BR,),
        in_specs=[
            pl.BlockSpec((BR, C), lambda i: (i, 0)),
            pl.BlockSpec((C, 128), lambda i: (0, 0)),
        ],
        out_specs=[
            pl.BlockSpec((BR, 128), lambda i: (i, 0)),
            pl.BlockSpec((BR, 128), lambda i: (i, 0)),
            pl.BlockSpec((1, 128), lambda i: (0, 0)),
            pl.BlockSpec((1, 128), lambda i: (0, 0)),
        ],
        out_shape=[
            jax.ShapeDtypeStruct((N, 128), jnp.int32),
            jax.ShapeDtypeStruct((N, 128), jnp.float32),
            jax.ShapeDtypeStruct((1, 128), jnp.float32),
            jax.ShapeDtypeStruct((1, 128), jnp.float32),
        ],
    )(xf, wr_pad)


def _mlp_body(be_ref, xs_ref, w1_ref, b1_ref, w2_ref, b2_ref, ys_ref):
    x = xs_ref[...]
    h = jnp.dot(x, w1_ref[0], preferred_element_type=jnp.float32) + b1_ref[0]
    h = 0.5 * h * (1.0 + jax.lax.erf(h * 0.7071067811865476))
    ys_ref[...] = (
        jnp.dot(h, w2_ref[0], preferred_element_type=jnp.float32) + b2_ref[0])


def _mlp(block_expert, xs, w1, b1, w2, b2):
    grid_spec = pltpu.PrefetchScalarGridSpec(
        num_scalar_prefetch=1,
        grid=(NB,),
        in_specs=[
            pl.BlockSpec((BLK, C), lambda b, be: (b, 0)),
            pl.BlockSpec((1, C, H), lambda b, be: (be[b], 0, 0)),
            pl.BlockSpec((1, 1, H), lambda b, be: (be[b], 0, 0)),
            pl.BlockSpec((1, H, C), lambda b, be: (be[b], 0, 0)),
            pl.BlockSpec((1, 1, C), lambda b, be: (be[b], 0, 0)),
        ],
        out_specs=pl.BlockSpec((BLK, C), lambda b, be: (b, 0)),
    )
    return pl.pallas_call(
        _mlp_body,
        grid_spec=grid_spec,
        out_shape=jax.ShapeDtypeStruct((PAD_CAP, C), jnp.float32),
    )(block_expert, xs, w1, b1, w2, b2)


def _sc_gather(xf, row_ids):
    ncheck = GRW // GCH

    @functools.partial(
        pl.kernel,
        mesh=_sc_mesh(),
        out_type=jax.ShapeDtypeStruct((PAD_CAP, C), jnp.float32),
        scratch_types=[
            pltpu.VMEM((GRW,), jnp.int32),
            [pltpu.VMEM((GCH, C), jnp.float32)] * GNB,
            [pltpu.SemaphoreType.DMA] * GNB,
            [pltpu.SemaphoreType.DMA] * GNB,
            [pltpu.SemaphoreType.DMA] * GNB,
        ],
    )
    def k(xf_hbm, ids_hbm, out_hbm, idx_v, bufs, gsems, g2sems, wsems):
        wid = lax.axis_index("s") * 2 + lax.axis_index("c")
        base = wid * GRW
        pltpu.sync_copy(ids_hbm.at[pl.ds(base, GRW)], idx_v)
        gh = GCH // 2

        def gath(ci):
            s = ci % GNB
            return (
                pltpu.async_copy(
                    xf_hbm.at[idx_v.at[pl.ds(ci * GCH, gh)]],
                    bufs[s].at[pl.ds(0, gh)], gsems[s]),
                pltpu.async_copy(
                    xf_hbm.at[idx_v.at[pl.ds(ci * GCH + gh, gh)]],
                    bufs[s].at[pl.ds(gh, gh)], g2sems[s]),
            )

        def wrb(ci):
            return pltpu.async_copy(
                bufs[ci % GNB],
                out_hbm.at[pl.ds(base + ci * GCH, GCH)], wsems[ci % GNB])

        gds = [None] * ncheck
        wds = [None] * ncheck
        gds[0] = gath(0)
        if ncheck > 1:
            gds[1] = gath(1)
        for ci in range(ncheck):
            gds[ci][0].wait()
            gds[ci][1].wait()
            wds[ci] = wrb(ci)
            nxt = ci + 2
            if nxt < ncheck:
                if nxt >= GNB:
                    wds[nxt - GNB].wait()
                gds[nxt] = gath(nxt)
        for ci in range(max(0, ncheck - GNB), ncheck):
            if wds[ci] is not None:
                wds[ci].wait()

    return k(xf, row_ids)


def _sc_combine(ys, pos0, pos1, w0, w1):
    ncheck = CW // CCH

    @functools.partial(
        pl.kernel,
        mesh=_sc_mesh(),
        out_type=jax.ShapeDtypeStruct((N, C), jnp.float32),
        scratch_types=[
            pltpu.VMEM((CW,), jnp.int32),
            pltpu.VMEM((CW,), jnp.int32),
            pltpu.VMEM((CCH, 16), jnp.float32),
            pltpu.VMEM((CCH, 16), jnp.float32),
            [pltpu.VMEM((CCH, C), jnp.float32)] * CNB,
            [pltpu.VMEM((CCH, C), jnp.float32)] * CNB,
            [pltpu.SemaphoreType.DMA] * CNB,
            [pltpu.SemaphoreType.DMA] * CNB,
            [pltpu.SemaphoreType.DMA] * CNB,
        ],
    )
    def k(ys_hbm, p0_hbm, p1_hbm, w0_hbm, w1_hbm, out_hbm,
          i0_v, i1_v, w0_v, w1_v, a0s, a1s, g0sems, g1sems, wsems):
        wid = lax.axis_index("s") * 2 + lax.axis_index("c")
        base = wid * CW
        pltpu.sync_copy(p0_hbm.at[pl.ds(base, CW)], i0_v)
        pltpu.sync_copy(p1_hbm.at[pl.ds(base, CW)], i1_v)

        def gath(ci):
            s = ci % CNB
            return (
                pltpu.async_copy(
                    ys_hbm.at[i0_v.at[pl.ds(ci * CCH, CCH)]], a0s[s],
                    g0sems[s]),
                pltpu.async_copy(
                    ys_hbm.at[i1_v.at[pl.ds(ci * CCH, CCH)]], a1s[s],
                    g1sems[s]),
            )

        def wrb(ci):
            s = ci % CNB
            return pltpu.async_copy(
                a0s[s], out_hbm.at[pl.ds(base + ci * CCH, CCH)], wsems[s])

        gds = [None] * ncheck
        wds = [None] * ncheck
        gds[0] = gath(0)
        if ncheck > 1:
            gds[1] = gath(1)
        for ci in range(ncheck):
            s = ci % CNB
            gds[ci][0].wait()
            gds[ci][1].wait()
            a0_v = a0s[s]
            a1_v = a1s[s]
            pltpu.sync_copy(w0_hbm.at[pl.ds(base + ci * CCH, CCH)], w0_v)
            pltpu.sync_copy(w1_hbm.at[pl.ds(base + ci * CCH, CCH)], w1_v)

            def row_loop(r, carry2, a0_v=a0_v, a1_v=a1_v):
                w0s = w0_v[r, :]
                w1s = w1_v[r, :]

                def col_loop(j, carry3):
                    col = j * 64
                    for u in range(4):
                        cu = col + u * 16
                        a0_v[r, pl.ds(cu, 16)] = (
                            a0_v[r, pl.ds(cu, 16)] * w0s
                            + a1_v[r, pl.ds(cu, 16)] * w1s)
                    return carry3

                return lax.fori_loop(0, C // 64, col_loop, carry2)

            lax.fori_loop(0, CCH, row_loop, 0)
            wds[ci] = wrb(ci)
            nxt = ci + 2
            if nxt < ncheck:
                if nxt >= CNB:
                    wds[nxt - CNB].wait()
                gds[nxt] = gath(nxt)
        for ci in range(max(0, ncheck - CNB), ncheck):
            if wds[ci] is not None:
                wds[ci].wait()

    return k(ys, pos0, pos1, w0, w1)


def kernel(x, Wr, W1, b1, W2, b2):
    xf = x.reshape(-1, C)
    wr_pad = jnp.pad(Wr, ((0, 0), (0, 128 - E)))
    idx_out, w_out, util_out, cnt_out = _router(xf, wr_pad)
    util = util_out[0, :E]

    e_flat = idx_out[:, :K].reshape(-1)
    rank = idx_out[:, K:2 * K].reshape(-1)
    cnt = cnt_out[0, :E].astype(jnp.int32)
    cnt_pad = ((cnt + BLK - 1) // BLK) * BLK
    ends = jnp.cumsum(cnt_pad)
    start = ends - cnt_pad
    p = start[e_flat] + rank
    row_ids = jnp.zeros((PAD_CAP,), jnp.int32).at[p].set(
        jnp.arange(NPAIR, dtype=jnp.int32) // K)
    pos = p.reshape(N, K)
    bstart = jnp.arange(NB, dtype=jnp.int32) * BLK
    block_expert = jnp.minimum(
        jnp.sum((bstart[:, None] >= ends[None, :]).astype(jnp.int32), axis=1),
        E - 1).astype(jnp.int32)

    xs = _sc_gather(xf, row_ids)

    ys = _mlp(block_expert, xs, W1, b1.reshape(E, 1, H), W2,
              b2.reshape(E, 1, C))

    w0_wide = jnp.broadcast_to(w_out[:, 0:1], (N, 16))
    w1_wide = jnp.broadcast_to(w_out[:, 1:2], (N, 16))
    out = _sc_combine(ys, pos[:, 0], pos[:, 1], w0_wide, w1_wide)
    return out.reshape(x.shape), util


# speedup vs baseline: 1.1216x; 1.1216x over previous
import functools

import jax
import jax.numpy as jnp
from jax import lax
from jax.experimental import pallas as pl
from jax.experimental.pallas import tpu as pltpu
from jax.experimental.pallas import tpu_sc as plsc

N = 4096
C = 1024
H = 2048
E = 8
K = 2
BLK = 128
NPAIR = N * K
NB = NPAIR // BLK + E
PAD_CAP = NB * BLK
BR = 512
NEG = -1e30

NW = 32
GRW = PAD_CAP // NW
GCH = 32
GNB = 3
CW = N // NW
CCH = 16
CNB = 3

def _sc_mesh():
    return plsc.VectorSubcoreMesh(core_axis_name="c", subcore_axis_name="s")


def _router_body(x_ref, wr_ref, idx_ref, w_ref, util_ref, cnt_ref):
    step = pl.program_id(0)
    x = x_ref[...]
    logits = jnp.dot(x, wr_ref[...], preferred_element_type=jnp.float32)
    lane = jax.lax.broadcasted_iota(jnp.int32, (BR, 128), 1)
    valid = lane < E
    lg = jnp.where(valid, logits, NEG)
    m1 = jnp.max(lg, axis=1, keepdims=True)
    a1 = jnp.min(jnp.where(lg == m1, lane, 128), axis=1, keepdims=True)
    lg2 = jnp.where(lane == a1, NEG, lg)
    m2 = jnp.max(lg2, axis=1, keepdims=True)
    a2 = jnp.min(jnp.where(lg2 == m2, lane, 128), axis=1, keepdims=True)
    ex = jnp.where(valid, jnp.exp(lg - m1), 0.0)
    z = jnp.sum(ex, axis=1, keepdims=True)
    p1 = 1.0 / z
    p2 = jnp.exp(m2 - m1) / z
    denom = p1 + p2 + 1e-9
    w1 = p1 / denom
    w2 = p2 / denom
    w_ref[...] = jnp.where(lane == 0, w1, jnp.where(lane == 1, w2, 0.0))

    sel1 = (lane == a1).astype(jnp.int32)
    sel2 = (lane == a2).astype(jnp.int32)
    both = sel1 + sel2
    acc = both
    d = 1
    while d < BR:
        top = jnp.zeros((d, 128), jnp.int32)
        acc = acc + jnp.concatenate([top, acc[:BR - d]], axis=0)
        d *= 2
    cum_ex = acc - both

    @pl.when(step == 0)
    def _():
        util_ref[...] = jnp.zeros_like(util_ref)
        cnt_ref[...] = jnp.zeros_like(cnt_ref)

    carry = cnt_ref[...].astype(jnp.int32)
    rank_base = cum_ex + carry
    rank0 = jnp.sum(jnp.where(lane == a1, rank_base, 0), axis=1, keepdims=True)
    rank1 = jnp.sum(jnp.where(lane == a2, rank_base, 0), axis=1, keepdims=True)
    idx_ref[...] = jnp.where(
        lane == 0, a1,
        jnp.where(lane == 1, a2,
                  jnp.where(lane == 2, rank0,
                            jnp.where(lane == 3, rank1, 0))))

    cnt = jnp.sum(both.astype(jnp.float32), axis=0, keepdims=True)
    util_ref[...] += cnt
    cnt_ref[...] += cnt

    @pl.when(step == (N // BR) - 1)
    def _():
        c = util_ref[...]
        util_ref[...] = c / (jnp.sum(c) + 1e-9)


def _router(xf, wr_pad):
    return pl.pallas_call(
        _router_body,
        grid=(N // BR,),
        in_specs=[
            pl.BlockSpec((BR, C), lambda i: (i, 0)),
            pl.BlockSpec((C, 128), lambda i: (0, 0)),
        ],
        out_specs=[
            pl.BlockSpec((BR, 128), lambda i: (i, 0)),
            pl.BlockSpec((BR, 128), lambda i: (i, 0)),
            pl.BlockSpec((1, 128), lambda i: (0, 0)),
            pl.BlockSpec((1, 128), lambda i: (0, 0)),
        ],
        out_shape=[
            jax.ShapeDtypeStruct((N, 128), jnp.int32),
            jax.ShapeDtypeStruct((N, 128), jnp.float32),
            jax.ShapeDtypeStruct((1, 128), jnp.float32),
            jax.ShapeDtypeStruct((1, 128), jnp.float32),
        ],
    )(xf, wr_pad)


def _mlp_body(be_ref, xs_ref, w1_ref, b1_ref, w2_ref, b2_ref, ys_ref):
    x = xs_ref[...]
    h = jnp.dot(x, w1_ref[0], preferred_element_type=jnp.float32) + b1_ref[0]
    h = 0.5 * h * (1.0 + jax.lax.erf(h * 0.7071067811865476))
    ys_ref[...] = (
        jnp.dot(h, w2_ref[0], preferred_element_type=jnp.float32) + b2_ref[0])


def _mlp(block_expert, xs, w1, b1, w2, b2):
    grid_spec = pltpu.PrefetchScalarGridSpec(
        num_scalar_prefetch=1,
        grid=(NB,),
        in_specs=[
            pl.BlockSpec((BLK, C), lambda b, be: (b, 0)),
            pl.BlockSpec((1, C, H), lambda b, be: (be[b], 0, 0)),
            pl.BlockSpec((1, 1, H), lambda b, be: (be[b], 0, 0)),
            pl.BlockSpec((1, H, C), lambda b, be: (be[b], 0, 0)),
            pl.BlockSpec((1, 1, C), lambda b, be: (be[b], 0, 0)),
        ],
        out_specs=pl.BlockSpec((BLK, C), lambda b, be: (b, 0)),
    )
    return pl.pallas_call(
        _mlp_body,
        grid_spec=grid_spec,
        out_shape=jax.ShapeDtypeStruct((PAD_CAP, C), jnp.float32),
    )(block_expert, xs, w1, b1, w2, b2)


def _sc_gather(xf, row_ids):
    ncheck = GRW // GCH

    @functools.partial(
        pl.kernel,
        mesh=_sc_mesh(),
        out_type=jax.ShapeDtypeStruct((PAD_CAP, C), jnp.float32),
        scratch_types=[
            pltpu.VMEM((GRW,), jnp.int32),
            [pltpu.VMEM((GCH, C), jnp.float32)] * GNB,
            [pltpu.SemaphoreType.DMA] * GNB,
            [pltpu.SemaphoreType.DMA] * GNB,
            [pltpu.SemaphoreType.DMA] * GNB,
        ],
    )
    def k(xf_hbm, ids_hbm, out_hbm, idx_v, bufs, gsems, g2sems, wsems):
        wid = lax.axis_index("s") * 2 + lax.axis_index("c")
        base = wid * GRW
        pltpu.sync_copy(ids_hbm.at[pl.ds(base, GRW)], idx_v)
        gh = GCH // 2

        def gath(ci):
            s = ci % GNB
            return (
                pltpu.async_copy(
                    xf_hbm.at[idx_v.at[pl.ds(ci * GCH, gh)]],
                    bufs[s].at[pl.ds(0, gh)], gsems[s]),
                pltpu.async_copy(
                    xf_hbm.at[idx_v.at[pl.ds(ci * GCH + gh, gh)]],
                    bufs[s].at[pl.ds(gh, gh)], g2sems[s]),
            )

        def wrb(ci):
            return pltpu.async_copy(
                bufs[ci % GNB],
                out_hbm.at[pl.ds(base + ci * GCH, GCH)], wsems[ci % GNB])

        gds = [None] * ncheck
        wds = [None] * ncheck
        gds[0] = gath(0)
        if ncheck > 1:
            gds[1] = gath(1)
        for ci in range(ncheck):
            gds[ci][0].wait()
            gds[ci][1].wait()
            wds[ci] = wrb(ci)
            nxt = ci + 2
            if nxt < ncheck:
                if nxt >= GNB:
                    wds[nxt - GNB].wait()
                gds[nxt] = gath(nxt)
        for ci in range(max(0, ncheck - GNB), ncheck):
            if wds[ci] is not None:
                wds[ci].wait()

    return k(xf, row_ids)


def _sc_combine(ys, pos0, pos1, w0, w1):
    ncheck = CW // CCH

    @functools.partial(
        pl.kernel,
        mesh=_sc_mesh(),
        out_type=jax.ShapeDtypeStruct((N, C), jnp.float32),
        scratch_types=[
            pltpu.VMEM((CW,), jnp.int32),
            pltpu.VMEM((CW,), jnp.int32),
            pltpu.VMEM((CCH, 16), jnp.float32),
            pltpu.VMEM((CCH, 16), jnp.float32),
            [pltpu.VMEM((CCH, C), jnp.float32)] * CNB,
            [pltpu.VMEM((CCH, C), jnp.float32)] * CNB,
            [pltpu.SemaphoreType.DMA] * CNB,
            [pltpu.SemaphoreType.DMA] * CNB,
            [pltpu.SemaphoreType.DMA] * CNB,
        ],
    )
    def k(ys_hbm, p0_hbm, p1_hbm, w0_hbm, w1_hbm, out_hbm,
          i0_v, i1_v, w0_v, w1_v, a0s, a1s, g0sems, g1sems, wsems):
        wid = lax.axis_index("s") * 2 + lax.axis_index("c")
        base = wid * CW
        pltpu.sync_copy(p0_hbm.at[pl.ds(base, CW)], i0_v)
        pltpu.sync_copy(p1_hbm.at[pl.ds(base, CW)], i1_v)

        def gath(ci):
            s = ci % CNB
            return (
                pltpu.async_copy(
                    ys_hbm.at[i0_v.at[pl.ds(ci * CCH, CCH)]], a0s[s],
                    g0sems[s]),
                pltpu.async_copy(
                    ys_hbm.at[i1_v.at[pl.ds(ci * CCH, CCH)]], a1s[s],
                    g1sems[s]),
            )

        def wrb(ci):
            s = ci % CNB
            return pltpu.async_copy(
                a0s[s], out_hbm.at[pl.ds(base + ci * CCH, CCH)], wsems[s])

        gds = [None] * ncheck
        wds = [None] * ncheck
        gds[0] = gath(0)
        if ncheck > 1:
            gds[1] = gath(1)
        for ci in range(ncheck):
            s = ci % CNB
            gds[ci][0].wait()
            gds[ci][1].wait()
            a0_v = a0s[s]
            a1_v = a1s[s]
            pltpu.sync_copy(w0_hbm.at[pl.ds(base + ci * CCH, CCH)], w0_v)
            pltpu.sync_copy(w1_hbm.at[pl.ds(base + ci * CCH, CCH)], w1_v)

            def row_loop(r, carry2, a0_v=a0_v, a1_v=a1_v):
                w0s = w0_v[r, :]
                w1s = w1_v[r, :]

                def col_loop(j, carry3):
                    col = j * 64
                    for u in range(4):
                        cu = col + u * 16
                        a0_v[r, pl.ds(cu, 16)] = (
                            a0_v[r, pl.ds(cu, 16)] * w0s
                            + a1_v[r, pl.ds(cu, 16)] * w1s)
                    return carry3

                return lax.fori_loop(0, C // 64, col_loop, carry2)

            lax.fori_loop(0, CCH, row_loop, 0)
            wds[ci] = wrb(ci)
            nxt = ci + 2
            if nxt < ncheck:
                if nxt >= CNB:
                    wds[nxt - CNB].wait()
                gds[nxt] = gath(nxt)
        for ci in range(max(0, ncheck - CNB), ncheck):
            if wds[ci] is not None:
                wds[ci].wait()

    return k(ys, pos0, pos1, w0, w1)


def kernel(x, Wr, W1, b1, W2, b2):
    xf = x.reshape(-1, C)
    wr_pad = jnp.pad(Wr, ((0, 0), (0, 128 - E)))
    idx_out, w_out, util_out, cnt_out = _router(xf, wr_pad)
    util = util_out[0, :E]

    e_flat = idx_out[:, :K].reshape(-1)
    rank = idx_out[:, K:2 * K].reshape(-1)
    cnt = cnt_out[0, :E].astype(jnp.int32)
    cnt_pad = ((cnt + BLK - 1) // BLK) * BLK
    ends = jnp.cumsum(cnt_pad)
    start = ends - cnt_pad
    p = start[e_flat] + rank
    row_ids = jnp.zeros((PAD_CAP,), jnp.int32).at[p].set(
        jnp.arange(NPAIR, dtype=jnp.int32) // K)
    pos = p.reshape(N, K)
    bstart = jnp.arange(NB, dtype=jnp.int32) * BLK
    block_expert = jnp.minimum(
        jnp.sum((bstart[:, None] >= ends[None, :]).astype(jnp.int32), axis=1),
        E - 1).astype(jnp.int32)

    xs = _sc_gather(xf, row_ids)

    ys = _mlp(block_expert, xs, W1, b1.reshape(E, 1, H), W2,
              b2.reshape(E, 1, C))

    w0_wide = jnp.broadcast_to(w_out[:, 0:1], (N, 16))
    w1_wide = jnp.broadcast_to(w_out[:, 1:2], (N, 16))
    out = _sc_combine(ys, pos[:, 0], pos[:, 1], w0_wide, w1_wide)
    return out.reshape(x.shape), util


# speedup vs baseline: 1.5941x; 1.4213x over previous
import functools

import jax
import jax.numpy as jnp
from jax import lax
from jax.experimental import pallas as pl
from jax.experimental.pallas import tpu as pltpu
from jax.experimental.pallas import tpu_sc as plsc

N = 4096
C = 1024
H = 2048
E = 8
K = 2
BLK = 128
NPAIR = N * K
NB = NPAIR // BLK + E
PAD_CAP = NB * BLK
BR = 512
NEG = -1e30

NW = 32
CW = N // NW
SCH = 32
SNB = 3
CCH = 16
CNB = 3

def _sc_mesh():
    return plsc.VectorSubcoreMesh(core_axis_name="c", subcore_axis_name="s")


def _router_body(x_ref, wr_ref, idx_ref, w_ref, util_ref, cnt_ref):
    step = pl.program_id(0)
    x = x_ref[...]
    logits = jnp.dot(x, wr_ref[...], preferred_element_type=jnp.float32)
    lane = jax.lax.broadcasted_iota(jnp.int32, (BR, 128), 1)
    valid = lane < E
    lg = jnp.where(valid, logits, NEG)
    m1 = jnp.max(lg, axis=1, keepdims=True)
    a1 = jnp.min(jnp.where(lg == m1, lane, 128), axis=1, keepdims=True)
    lg2 = jnp.where(lane == a1, NEG, lg)
    m2 = jnp.max(lg2, axis=1, keepdims=True)
    a2 = jnp.min(jnp.where(lg2 == m2, lane, 128), axis=1, keepdims=True)
    ex = jnp.where(valid, jnp.exp(lg - m1), 0.0)
    z = jnp.sum(ex, axis=1, keepdims=True)
    p1 = 1.0 / z
    p2 = jnp.exp(m2 - m1) / z
    denom = p1 + p2 + 1e-9
    w1 = p1 / denom
    w2 = p2 / denom
    w_ref[...] = jnp.where(lane == 0, w1, jnp.where(lane == 1, w2, 0.0))

    sel1 = (lane == a1).astype(jnp.int32)
    sel2 = (lane == a2).astype(jnp.int32)
    both = sel1 + sel2
    acc = both
    d = 1
    while d < BR:
        top = jnp.zeros((d, 128), jnp.int32)
        acc = acc + jnp.concatenate([top, acc[:BR - d]], axis=0)
        d *= 2
    cum_ex = acc - both

    @pl.when(step == 0)
    def _():
        util_ref[...] = jnp.zeros_like(util_ref)
        cnt_ref[...] = jnp.zeros_like(cnt_ref)

    carry = cnt_ref[...].astype(jnp.int32)
    rank_base = cum_ex + carry
    rank0 = jnp.sum(jnp.where(lane == a1, rank_base, 0), axis=1, keepdims=True)
    rank1 = jnp.sum(jnp.where(lane == a2, rank_base, 0), axis=1, keepdims=True)
    idx_ref[...] = jnp.where(
        lane == 0, a1,
        jnp.where(lane == 1, a2,
                  jnp.where(lane == 2, rank0,
                            jnp.where(lane == 3, rank1, 0))))

    cnt = jnp.sum(both.astype(jnp.float32), axis=0, keepdims=True)
    util_ref[...] += cnt
    cnt_ref[...] += cnt

    @pl.when(step == (N // BR) - 1)
    def _():
        c = util_ref[...]
        util_ref[...] = c / (jnp.sum(c) + 1e-9)


def _router(xf, wr_pad):
    return pl.pallas_call(
        _router_body,
        grid=(N // BR,),
        in_specs=[
            pl.BlockSpec((BR, C), lambda i: (i, 0)),
            pl.BlockSpec((C, 128), lambda i: (0, 0)),
        ],
        out_specs=[
            pl.BlockSpec((BR, 128), lambda i: (i, 0)),
            pl.BlockSpec((BR, 128), lambda i: (i, 0)),
            pl.BlockSpec((1, 128), lambda i: (0, 0)),
            pl.BlockSpec((1, 128), lambda i: (0, 0)),
        ],
        out_shape=[
            jax.ShapeDtypeStruct((N, 128), jnp.int32),
            jax.ShapeDtypeStruct((N, 128), jnp.float32),
            jax.ShapeDtypeStruct((1, 128), jnp.float32),
            jax.ShapeDtypeStruct((1, 128), jnp.float32),
        ],
    )(xf, wr_pad)


def _mlp_body(be_ref, xs_ref, w1_ref, b1_ref, w2_ref, b2_ref, ys_ref):
    x = xs_ref[...]
    h = jnp.dot(x, w1_ref[0], preferred_element_type=jnp.float32) + b1_ref[0]
    h = 0.5 * h * (1.0 + jax.lax.erf(h * 0.7071067811865476))
    ys_ref[...] = (
        jnp.dot(h, w2_ref[0], preferred_element_type=jnp.float32) + b2_ref[0])


def _mlp(block_expert, xs, w1, b1, w2, b2):
    grid_spec = pltpu.PrefetchScalarGridSpec(
        num_scalar_prefetch=1,
        grid=(NB,),
        in_specs=[
            pl.BlockSpec((BLK, C), lambda b, be: (b, 0)),
            pl.BlockSpec((1, C, H), lambda b, be: (be[b], 0, 0)),
            pl.BlockSpec((1, 1, H), lambda b, be: (be[b], 0, 0)),
            pl.BlockSpec((1, H, C), lambda b, be: (be[b], 0, 0)),
            pl.BlockSpec((1, 1, C), lambda b, be: (be[b], 0, 0)),
        ],
        out_specs=pl.BlockSpec((BLK, C), lambda b, be: (b, 0)),
    )
    return pl.pallas_call(
        _mlp_body,
        grid_spec=grid_spec,
        out_shape=jax.ShapeDtypeStruct((PAD_CAP, C), jnp.float32),
    )(block_expert, xs, w1, b1, w2, b2)


def _sc_dispatch(xf, p0, p1):
    ncheck = CW // SCH

    @functools.partial(
        pl.kernel,
        mesh=_sc_mesh(),
        out_type=jax.ShapeDtypeStruct((PAD_CAP, C), jnp.float32),
        scratch_types=[
            [pltpu.VMEM((SCH,), jnp.int32)] * SNB,
            [pltpu.VMEM((SCH,), jnp.int32)] * SNB,
            [pltpu.VMEM((SCH, C), jnp.float32)] * SNB,
            [pltpu.SemaphoreType.DMA] * SNB,
            [pltpu.SemaphoreType.DMA] * SNB,
            [pltpu.SemaphoreType.DMA] * SNB,
        ],
    )
    def k(xf_hbm, p0_hbm, p1_hbm, out_hbm, i0s, i1s, bufs, rsems, s0sems,
          s1sems):
        wid = lax.axis_index("s") * 2 + lax.axis_index("c")
        base = wid * CW

        def rd(ci):
            s = ci % SNB
            pltpu.sync_copy(p0_hbm.at[pl.ds(base + ci * SCH, SCH)], i0s[s])
            pltpu.sync_copy(p1_hbm.at[pl.ds(base + ci * SCH, SCH)], i1s[s])
            return pltpu.async_copy(
                xf_hbm.at[pl.ds(base + ci * SCH, SCH)], bufs[s], rsems[s])

        def sct(ci):
            s = ci % SNB
            return (
                pltpu.async_copy(bufs[s], out_hbm.at[i0s[s]], s0sems[s]),
                pltpu.async_copy(bufs[s], out_hbm.at[i1s[s]], s1sems[s]),
            )

        rds = [None] * ncheck
        wds = [None] * ncheck
        rds[0] = rd(0)
        if ncheck > 1:
            rds[1] = rd(1)
        for ci in range(ncheck):
            rds[ci].wait()
            wds[ci] = sct(ci)
            nxt = ci + 2
            if nxt < ncheck:
                if nxt >= SNB:
                    wds[nxt - SNB][0].wait()
                    wds[nxt - SNB][1].wait()
                rds[nxt] = rd(nxt)
        for ci in range(max(0, ncheck - SNB), ncheck):
            if wds[ci] is not None:
                wds[ci][0].wait()
                wds[ci][1].wait()

    return k(xf, p0, p1)


def _sc_combine(ys, pos0, pos1, w0, w1):
    ncheck = CW // CCH

    @functools.partial(
        pl.kernel,
        mesh=_sc_mesh(),
        out_type=jax.ShapeDtypeStruct((N, C), jnp.float32),
        scratch_types=[
            pltpu.VMEM((CW,), jnp.int32),
            pltpu.VMEM((CW,), jnp.int32),
            pltpu.VMEM((CCH, 16), jnp.float32),
            pltpu.VMEM((CCH, 16), jnp.float32),
            [pltpu.VMEM((CCH, C), jnp.float32)] * CNB,
            [pltpu.VMEM((CCH, C), jnp.float32)] * CNB,
            [pltpu.SemaphoreType.DMA] * CNB,
            [pltpu.SemaphoreType.DMA] * CNB,
            [pltpu.SemaphoreType.DMA] * CNB,
        ],
    )
    def k(ys_hbm, p0_hbm, p1_hbm, w0_hbm, w1_hbm, out_hbm,
          i0_v, i1_v, w0_v, w1_v, a0s, a1s, g0sems, g1sems, wsems):
        wid = lax.axis_index("s") * 2 + lax.axis_index("c")
        base = wid * CW
        pltpu.sync_copy(p0_hbm.at[pl.ds(base, CW)], i0_v)
        pltpu.sync_copy(p1_hbm.at[pl.ds(base, CW)], i1_v)

        def gath(ci):
            s = ci % CNB
            return (
                pltpu.async_copy(
                    ys_hbm.at[i0_v.at[pl.ds(ci * CCH, CCH)]], a0s[s],
                    g0sems[s]),
                pltpu.async_copy(
                    ys_hbm.at[i1_v.at[pl.ds(ci * CCH, CCH)]], a1s[s],
                    g1sems[s]),
            )

        def wrb(ci):
            s = ci % CNB
            return pltpu.async_copy(
                a0s[s], out_hbm.at[pl.ds(base + ci * CCH, CCH)], wsems[s])

        gds = [None] * ncheck
        wds = [None] * ncheck
        gds[0] = gath(0)
        if ncheck > 1:
            gds[1] = gath(1)
        for ci in range(ncheck):
            s = ci % CNB
            gds[ci][0].wait()
            gds[ci][1].wait()
            a0_v = a0s[s]
            a1_v = a1s[s]
            pltpu.sync_copy(w0_hbm.at[pl.ds(base + ci * CCH, CCH)], w0_v)
            pltpu.sync_copy(w1_hbm.at[pl.ds(base + ci * CCH, CCH)], w1_v)

            def row_loop(r, carry2, a0_v=a0_v, a1_v=a1_v):
                w0s = w0_v[r, :]
                w1s = w1_v[r, :]

                def col_loop(j, carry3):
                    col = j * 64
                    for u in range(4):
                        cu = col + u * 16
                        a0_v[r, pl.ds(cu, 16)] = (
                            a0_v[r, pl.ds(cu, 16)] * w0s
                            + a1_v[r, pl.ds(cu, 16)] * w1s)
                    return carry3

                return lax.fori_loop(0, C // 64, col_loop, carry2)

            lax.fori_loop(0, CCH, row_loop, 0)
            wds[ci] = wrb(ci)
            nxt = ci + 2
            if nxt < ncheck:
                if nxt >= CNB:
                    wds[nxt - CNB].wait()
                gds[nxt] = gath(nxt)
        for ci in range(max(0, ncheck - CNB), ncheck):
            if wds[ci] is not None:
                wds[ci].wait()

    return k(ys, pos0, pos1, w0, w1)


def kernel(x, Wr, W1, b1, W2, b2):
    xf = x.reshape(-1, C)
    wr_pad = jnp.pad(Wr, ((0, 0), (0, 128 - E)))
    idx_out, w_out, util_out, cnt_out = _router(xf, wr_pad)
    util = util_out[0, :E]

    e_flat = idx_out[:, :K].reshape(-1)
    rank = idx_out[:, K:2 * K].reshape(-1)
    cnt = cnt_out[0, :E].astype(jnp.int32)
    cnt_pad = ((cnt + BLK - 1) // BLK) * BLK
    ends = jnp.cumsum(cnt_pad)
    start = ends - cnt_pad
    p = start[e_flat] + rank
    pos = p.reshape(N, K)
    bstart = jnp.arange(NB, dtype=jnp.int32) * BLK
    block_expert = jnp.minimum(
        jnp.sum((bstart[:, None] >= ends[None, :]).astype(jnp.int32), axis=1),
        E - 1).astype(jnp.int32)

    xs = _sc_dispatch(xf, pos[:, 0], pos[:, 1])

    ys = _mlp(block_expert, xs, W1, b1.reshape(E, 1, H), W2,
              b2.reshape(E, 1, C))

    w0_wide = jnp.broadcast_to(w_out[:, 0:1], (N, 16))
    w1_wide = jnp.broadcast_to(w_out[:, 1:2], (N, 16))
    out = _sc_combine(ys, pos[:, 0], pos[:, 1], w0_wide, w1_wide)
    return out.reshape(x.shape), util


# speedup vs baseline: 1.6262x; 1.0201x over previous
import functools

import jax
import jax.numpy as jnp
from jax import lax
from jax.experimental import pallas as pl
from jax.experimental.pallas import tpu as pltpu
from jax.experimental.pallas import tpu_sc as plsc

N = 4096
C = 1024
H = 2048
E = 8
K = 2
BLK = 128
NPAIR = N * K
NB = NPAIR // BLK + E
PAD_CAP = NB * BLK
BR = 512
NEG = -1e30

NW = 32
CW = N // NW
SCH = 32
SNB = 3
CCH = 16
CNB = 3

def _sc_mesh():
    return plsc.VectorSubcoreMesh(core_axis_name="c", subcore_axis_name="s")


def _router_body(x_ref, wr_ref, idx_ref, rank_ref, w0_ref, w1_ref, util_ref,
                 cnt_ref):
    step = pl.program_id(0)
    x = x_ref[...]
    logits = jnp.dot(x, wr_ref[...], preferred_element_type=jnp.float32)
    lane = jax.lax.broadcasted_iota(jnp.int32, (BR, 128), 1)
    valid = lane < E
    lg = jnp.where(valid, logits, NEG)
    m1 = jnp.max(lg, axis=1, keepdims=True)
    a1 = jnp.min(jnp.where(lg == m1, lane, 128), axis=1, keepdims=True)
    lg2 = jnp.where(lane == a1, NEG, lg)
    m2 = jnp.max(lg2, axis=1, keepdims=True)
    a2 = jnp.min(jnp.where(lg2 == m2, lane, 128), axis=1, keepdims=True)
    ex = jnp.where(valid, jnp.exp(lg - m1), 0.0)
    z = jnp.sum(ex, axis=1, keepdims=True)
    p1 = 1.0 / z
    p2 = jnp.exp(m2 - m1) / z
    denom = p1 + p2 + 1e-9
    w1 = p1 / denom
    w2 = p2 / denom
    w0_ref[...] = jnp.broadcast_to(w1, (BR, 16))
    w1_ref[...] = jnp.broadcast_to(w2, (BR, 16))

    sel1 = (lane == a1).astype(jnp.int32)
    sel2 = (lane == a2).astype(jnp.int32)
    both = sel1 + sel2
    acc = both
    d = 1
    while d < BR:
        top = jnp.zeros((d, 128), jnp.int32)
        acc = acc + jnp.concatenate([top, acc[:BR - d]], axis=0)
        d *= 2
    cum_ex = acc - both

    @pl.when(step == 0)
    def _():
        util_ref[...] = jnp.zeros_like(util_ref)
        cnt_ref[...] = jnp.zeros_like(cnt_ref)

    carry = cnt_ref[...].astype(jnp.int32)
    rank_base = cum_ex + carry
    rank0 = jnp.sum(jnp.where(lane == a1, rank_base, 0), axis=1, keepdims=True)
    rank1 = jnp.sum(jnp.where(lane == a2, rank_base, 0), axis=1, keepdims=True)
    idx_ref[...] = jnp.concatenate([a1, a2], axis=1)
    rank_ref[...] = jnp.concatenate([rank0, rank1], axis=1)

    cnt = jnp.sum(both.astype(jnp.float32), axis=0, keepdims=True)
    util_ref[...] += cnt
    cnt_ref[...] += cnt

    @pl.when(step == (N // BR) - 1)
    def _():
        c = util_ref[...]
        util_ref[...] = c / (jnp.sum(c) + 1e-9)


def _router(xf, wr_pad):
    return pl.pallas_call(
        _router_body,
        grid=(N // BR,),
        in_specs=[
            pl.BlockSpec((BR, C), lambda i: (i, 0)),
            pl.BlockSpec((C, 128), lambda i: (0, 0)),
        ],
        out_specs=[
            pl.BlockSpec((BR, 2), lambda i: (i, 0)),
            pl.BlockSpec((BR, 2), lambda i: (i, 0)),
            pl.BlockSpec((BR, 16), lambda i: (i, 0)),
            pl.BlockSpec((BR, 16), lambda i: (i, 0)),
            pl.BlockSpec((1, 128), lambda i: (0, 0)),
            pl.BlockSpec((1, 128), lambda i: (0, 0)),
        ],
        out_shape=[
            jax.ShapeDtypeStruct((N, 2), jnp.int32),
            jax.ShapeDtypeStruct((N, 2), jnp.int32),
            jax.ShapeDtypeStruct((N, 16), jnp.float32),
            jax.ShapeDtypeStruct((N, 16), jnp.float32),
            jax.ShapeDtypeStruct((1, 128), jnp.float32),
            jax.ShapeDtypeStruct((1, 128), jnp.float32),
        ],
    )(xf, wr_pad)


def _mlp_body(be_ref, xs_ref, w1_ref, b1_ref, w2_ref, b2_ref, ys_ref):
    x = xs_ref[...]
    h = jnp.dot(x, w1_ref[0], preferred_element_type=jnp.float32) + b1_ref[0]
    h = 0.5 * h * (1.0 + jax.lax.erf(h * 0.7071067811865476))
    ys_ref[...] = (
        jnp.dot(h, w2_ref[0], preferred_element_type=jnp.float32) + b2_ref[0])


def _mlp(block_expert, xs, w1, b1, w2, b2):
    grid_spec = pltpu.PrefetchScalarGridSpec(
        num_scalar_prefetch=1,
        grid=(NB,),
        in_specs=[
            pl.BlockSpec((BLK, C), lambda b, be: (b, 0)),
            pl.BlockSpec((1, C, H), lambda b, be: (be[b], 0, 0)),
            pl.BlockSpec((1, 1, H), lambda b, be: (be[b], 0, 0)),
            pl.BlockSpec((1, H, C), lambda b, be: (be[b], 0, 0)),
            pl.BlockSpec((1, 1, C), lambda b, be: (be[b], 0, 0)),
        ],
        out_specs=pl.BlockSpec((BLK, C), lambda b, be: (b, 0)),
    )
    return pl.pallas_call(
        _mlp_body,
        grid_spec=grid_spec,
        out_shape=jax.ShapeDtypeStruct((PAD_CAP, C), jnp.float32),
    )(block_expert, xs, w1, b1, w2, b2)


def _sc_dispatch(xf, p0, p1):
    ncheck = CW // SCH

    @functools.partial(
        pl.kernel,
        mesh=_sc_mesh(),
        out_type=jax.ShapeDtypeStruct((PAD_CAP, C), jnp.float32),
        scratch_types=[
            [pltpu.VMEM((SCH,), jnp.int32)] * SNB,
            [pltpu.VMEM((SCH,), jnp.int32)] * SNB,
            [pltpu.VMEM((SCH, C), jnp.float32)] * SNB,
            [pltpu.SemaphoreType.DMA] * SNB,
            [pltpu.SemaphoreType.DMA] * SNB,
            [pltpu.SemaphoreType.DMA] * SNB,
        ],
    )
    def k(xf_hbm, p0_hbm, p1_hbm, out_hbm, i0s, i1s, bufs, rsems, s0sems,
          s1sems):
        wid = lax.axis_index("s") * 2 + lax.axis_index("c")
        base = wid * CW

        def rd(ci):
            s = ci % SNB
            pltpu.sync_copy(p0_hbm.at[pl.ds(base + ci * SCH, SCH)], i0s[s])
            pltpu.sync_copy(p1_hbm.at[pl.ds(base + ci * SCH, SCH)], i1s[s])
            return pltpu.async_copy(
                xf_hbm.at[pl.ds(base + ci * SCH, SCH)], bufs[s], rsems[s])

        def sct(ci):
            s = ci % SNB
            return (
                pltpu.async_copy(bufs[s], out_hbm.at[i0s[s]], s0sems[s]),
                pltpu.async_copy(bufs[s], out_hbm.at[i1s[s]], s1sems[s]),
            )

        rds = [None] * ncheck
        wds = [None] * ncheck
        rds[0] = rd(0)
        if ncheck > 1:
            rds[1] = rd(1)
        for ci in range(ncheck):
            rds[ci].wait()
            wds[ci] = sct(ci)
            nxt = ci + 2
            if nxt < ncheck:
                if nxt >= SNB:
                    wds[nxt - SNB][0].wait()
                    wds[nxt - SNB][1].wait()
                rds[nxt] = rd(nxt)
        for ci in range(max(0, ncheck - SNB), ncheck):
            if wds[ci] is not None:
                wds[ci][0].wait()
                wds[ci][1].wait()

    return k(xf, p0, p1)


def _sc_combine(ys, pos0, pos1, w0, w1):
    ncheck = CW // CCH

    @functools.partial(
        pl.kernel,
        mesh=_sc_mesh(),
        out_type=jax.ShapeDtypeStruct((N, C), jnp.float32),
        scratch_types=[
            pltpu.VMEM((CW,), jnp.int32),
            pltpu.VMEM((CW,), jnp.int32),
            pltpu.VMEM((CCH, 16), jnp.float32),
            pltpu.VMEM((CCH, 16), jnp.float32),
            [pltpu.VMEM((CCH, C), jnp.float32)] * CNB,
            [pltpu.VMEM((CCH, C), jnp.float32)] * CNB,
            [pltpu.SemaphoreType.DMA] * CNB,
            [pltpu.SemaphoreType.DMA] * CNB,
            [pltpu.SemaphoreType.DMA] * CNB,
        ],
    )
    def k(ys_hbm, p0_hbm, p1_hbm, w0_hbm, w1_hbm, out_hbm,
          i0_v, i1_v, w0_v, w1_v, a0s, a1s, g0sems, g1sems, wsems):
        wid = lax.axis_index("s") * 2 + lax.axis_index("c")
        base = wid * CW
        pltpu.sync_copy(p0_hbm.at[pl.ds(base, CW)], i0_v)
        pltpu.sync_copy(p1_hbm.at[pl.ds(base, CW)], i1_v)

        def gath(ci):
            s = ci % CNB
            return (
                pltpu.async_copy(
                    ys_hbm.at[i0_v.at[pl.ds(ci * CCH, CCH)]], a0s[s],
                    g0sems[s]),
                pltpu.async_copy(
                    ys_hbm.at[i1_v.at[pl.ds(ci * CCH, CCH)]], a1s[s],
                    g1sems[s]),
            )

        def wrb(ci):
            s = ci % CNB
            return pltpu.async_copy(
                a0s[s], out_hbm.at[pl.ds(base + ci * CCH, CCH)], wsems[s])

        gds = [None] * ncheck
        wds = [None] * ncheck
        gds[0] = gath(0)
        if ncheck > 1:
            gds[1] = gath(1)
        for ci in range(ncheck):
            s = ci % CNB
            gds[ci][0].wait()
            gds[ci][1].wait()
            a0_v = a0s[s]
            a1_v = a1s[s]
            pltpu.sync_copy(w0_hbm.at[pl.ds(base + ci * CCH, CCH)], w0_v)
            pltpu.sync_copy(w1_hbm.at[pl.ds(base + ci * CCH, CCH)], w1_v)

            def row_loop(r, carry2, a0_v=a0_v, a1_v=a1_v):
                w0s = w0_v[r, :]
                w1s = w1_v[r, :]

                def col_loop(j, carry3):
                    col = j * 64
                    for u in range(4):
                        cu = col + u * 16
                        a0_v[r, pl.ds(cu, 16)] = (
                            a0_v[r, pl.ds(cu, 16)] * w0s
                            + a1_v[r, pl.ds(cu, 16)] * w1s)
                    return carry3

                return lax.fori_loop(0, C // 64, col_loop, carry2)

            lax.fori_loop(0, CCH, row_loop, 0)
            wds[ci] = wrb(ci)
            nxt = ci + 2
            if nxt < ncheck:
                if nxt >= CNB:
                    wds[nxt - CNB].wait()
                gds[nxt] = gath(nxt)
        for ci in range(max(0, ncheck - CNB), ncheck):
            if wds[ci] is not None:
                wds[ci].wait()

    return k(ys, pos0, pos1, w0, w1)


def kernel(x, Wr, W1, b1, W2, b2):
    xf = x.reshape(-1, C)
    wr_pad = jnp.pad(Wr, ((0, 0), (0, 128 - E)))
    idx_out, rank_out, w0s, w1s, util_out, cnt_out = _router(xf, wr_pad)
    util = util_out[0, :E]

    e_flat = idx_out.reshape(-1)
    rank = rank_out.reshape(-1)
    cnt = cnt_out[0, :E].astype(jnp.int32)
    cnt_pad = ((cnt + BLK - 1) // BLK) * BLK
    ends = jnp.cumsum(cnt_pad)
    start = ends - cnt_pad
    p = start[e_flat] + rank
    pos = p.reshape(N, K)
    bstart = jnp.arange(NB, dtype=jnp.int32) * BLK
    block_expert = jnp.minimum(
        jnp.sum((bstart[:, None] >= ends[None, :]).astype(jnp.int32), axis=1),
        E - 1).astype(jnp.int32)

    xs = _sc_dispatch(xf, pos[:, 0], pos[:, 1])

    ys = _mlp(block_expert, xs, W1, b1.reshape(E, 1, H), W2,
              b2.reshape(E, 1, C))

    out = _sc_combine(ys, pos[:, 0], pos[:, 1], w0s, w1s)
    return out.reshape(x.shape), util
